# baseline (speedup 1.0000x reference)
import functools

import jax
import jax.numpy as jnp
from jax import lax
from jax.experimental import pallas as pl
from jax.experimental.pallas import tpu as pltpu
from jax.experimental.pallas import tpu_sc as plsc

NC = 2
NS = 16
NW = NC * NS

NB = 16


def _build(B0, S, V, D):
    rows_per_w = B0 // NW
    n_iter = rows_per_w // NB
    n_pairs = n_iter // 2
    assert rows_per_w * NW == B0
    assert n_pairs * 2 * NB == rows_per_w

    mesh = plsc.VectorSubcoreMesh(core_axis_name="c", subcore_axis_name="s")
    NBT = B0 // 128 // NW

    @functools.partial(
        pl.kernel,
        mesh=mesh,
        out_type=jax.ShapeDtypeStruct((S, D // 8, B0 // 128, 8, 128),
                                      jnp.float32),
        scratch_types=[
            pltpu.VMEM((rows_per_w, S), jnp.int32),
            pltpu.VMEM((2, NB, S, D), jnp.float32),
            pltpu.VMEM((2, S, D // 8, 8, NB), jnp.float32),
            pltpu.SemaphoreType.DMA,
            pltpu.SemaphoreType.DMA,
            pltpu.SemaphoreType.DMA,
            pltpu.SemaphoreType.DMA,
        ],
        compiler_params=pltpu.CompilerParams(
            use_tc_tiling_on_sc=False, needs_layout_passes=False),
    )
    def k(idx_hbm, table_hbm, out_hbm, idx_all, rows, tbuf, gsem0, gsem1,
          ssem0, ssem1):
        wid = lax.axis_index("s") * NC + lax.axis_index("c")
        base = wid * rows_per_w

        pltpu.sync_copy(idx_hbm.at[pl.ds(base, rows_per_w)], idx_all)

        def fire_gathers(it, buf, gsem):
            for j in range(NB):
                pltpu.async_copy(
                    table_hbm.at[idx_all.at[it * NB + j]],
                    rows.at[buf, j],
                    gsem)

        def wait_gathers(buf, gsem):
            for j in range(NB):
                pltpu.make_async_copy(
                    table_hbm.at[idx_all.at[0]],
                    rows.at[buf, j],
                    gsem).wait()

        def fire_store(it, buf, ssem):
            b0 = base + it * NB
            bt = b0 // 128
            bl = b0 % 128
            pltpu.async_copy(
                tbuf.at[buf],
                out_hbm.at[:, :, bt, :, pl.ds(bl, NB)],
                ssem)

        def wait_store(buf, ssem):
            pltpu.make_async_copy(
                tbuf.at[buf], out_hbm.at[:, :, 0, :, pl.ds(0, NB)],
                ssem).wait()

        lane = lax.iota(jnp.int32, 16)

        def transpose_rows(buf):
            def srow(s, carry):
                sv = jnp.full((16,), s, jnp.int32)
                for c in range(D):
                    cv = jnp.full((16,), c, jnp.int32)
                    v = plsc.load_gather(rows.at[buf], [lane, sv, cv])
                    tbuf[buf, s, c // 8, c % 8] = v
                return carry
            lax.fori_loop(0, S, srow, 0)

        fire_gathers(0, 0, gsem0)

        def body(p, carry):
            it0 = 2 * p
            it1 = it0 + 1
            @pl.when(p > 0)
            def _():
                wait_store(1, ssem1)
            fire_gathers(it1, 1, gsem1)
            wait_gathers(0, gsem0)
            transpose_rows(0)
            fire_store(it0, 0, ssem0)
            wait_store(0, ssem0)
            @pl.when(p < n_pairs - 1)
            def _():
                fire_gathers(it0 + 2, 0, gsem0)
            wait_gathers(1, gsem1)
            transpose_rows(1)
            fire_store(it1, 1, ssem1)
            return carry

        lax.fori_loop(0, n_pairs, body, 0)
        wait_store(1, ssem1)

    return k


def kernel(token_ids, weights):
    B0, S = token_ids.shape
    V, D = weights.shape
    out5 = _build(B0, S, V, D)(token_ids, weights)
    return out5.transpose(2, 4, 0, 1, 3).reshape(B0, S, D)


# speedup vs baseline: 1.5764x; 1.5764x over previous
import functools

import jax
import jax.numpy as jnp
from jax import lax
from jax.experimental import pallas as pl
from jax.experimental.pallas import tpu as pltpu
from jax.experimental.pallas import tpu_sc as plsc

NC = 2
NS = 16
NW = NC * NS

NB = 16


def _build(B0, S, V, D):
    rows_per_w = B0 // NW
    n_iter = rows_per_w // NB
    n_pairs = n_iter // 2
    assert rows_per_w * NW == B0
    assert n_pairs * 2 * NB == rows_per_w

    mesh = plsc.VectorSubcoreMesh(core_axis_name="c", subcore_axis_name="s")

    @functools.partial(
        pl.kernel,
        mesh=mesh,
        out_type=jax.ShapeDtypeStruct((S, D // 8, B0 // 128, 8, 128),
                                      jnp.float32),
        scratch_types=[
            pltpu.VMEM((rows_per_w, S), jnp.int32),
            pltpu.VMEM((2, NB, S, D), jnp.float32),
            pltpu.VMEM((S, D // 8, 8, NB + 1), jnp.float32),
            pltpu.SemaphoreType.DMA,
            pltpu.SemaphoreType.DMA,
            pltpu.SemaphoreType.DMA,
        ],
        compiler_params=pltpu.CompilerParams(
            use_tc_tiling_on_sc=False, needs_layout_passes=False),
    )
    def k(idx_hbm, table_hbm, out_hbm, idx_all, rows, tbuf, gsem0, gsem1,
          ssem):
        wid = lax.axis_index("s") * NC + lax.axis_index("c")
        base = wid * rows_per_w

        pltpu.sync_copy(idx_hbm.at[pl.ds(base, rows_per_w)], idx_all)

        def fire_gathers(it, buf, gsem):
            for j in range(NB):
                pltpu.async_copy(
                    table_hbm.at[idx_all.at[it * NB + j]],
                    rows.at[buf, j],
                    gsem)

        def wait_gathers(buf, gsem):
            for j in range(NB):
                pltpu.make_async_copy(
                    table_hbm.at[idx_all.at[0]],
                    rows.at[buf, j],
                    gsem).wait()

        def fire_store(it):
            b0 = base + it * NB
            bt = b0 // 128
            bl = b0 % 128
            pltpu.async_copy(
                tbuf.at[:, :, :, pl.ds(0, NB)],
                out_hbm.at[:, :, bt, :, pl.ds(bl, NB)],
                ssem)

        def wait_store():
            pltpu.make_async_copy(
                tbuf.at[:, :, :, pl.ds(0, NB)],
                out_hbm.at[:, :, 0, :, pl.ds(0, NB)],
                ssem).wait()

        iot = lax.iota(jnp.int32, 16)
        ct_lo = iot // 8
        cs_lo = lax.rem(iot, 8)
        ct_hi = ct_lo + 2

        def transpose_rows(buf):
            def srow(s, carry):
                for j in range(NB):
                    jv = jnp.full((16,), j, jnp.int32)
                    sv = jnp.full((16,), s, jnp.int32)
                    v0 = rows[buf, j, s, pl.ds(0, 16)]
                    plsc.store_scatter(tbuf, [sv, ct_lo, cs_lo, jv], v0)
                    v1 = rows[buf, j, s, pl.ds(16, 16)]
                    plsc.store_scatter(tbuf, [sv, ct_hi, cs_lo, jv], v1)
                return carry
            lax.fori_loop(0, S, srow, 0)

        fire_gathers(0, 0, gsem0)

        def body(p, carry):
            it0 = 2 * p
            it1 = it0 + 1
            fire_gathers(it1, 1, gsem1)
            wait_gathers(0, gsem0)
            @pl.when(p > 0)
            def _():
                wait_store()
            transpose_rows(0)
            fire_store(it0)
            @pl.when(p < n_pairs - 1)
            def _():
                fire_gathers(it0 + 2, 0, gsem0)
            wait_gathers(1, gsem1)
            wait_store()
            transpose_rows(1)
            fire_store(it1)
            return carry

        lax.fori_loop(0, n_pairs, body, 0)
        wait_store()

    return k


def kernel(token_ids, weights):
    B0, S = token_ids.shape
    V, D = weights.shape
    out5 = _build(B0, S, V, D)(token_ids, weights)
    return out5.transpose(2, 4, 0, 1, 3).reshape(B0, S, D)
